# baseline (speedup 1.0000x reference)
import functools

import jax
import jax.numpy as jnp
from jax import lax
from jax.experimental import pallas as pl
from jax.experimental.pallas import tpu as pltpu
from jax.experimental.pallas import tpu_sc as plsc

N_NODES = 10000
N_PAD = 10112
N_EDGES = 320000
D = 128
NC = 2
NS = 16
NW = NC * NS
E_PER_W = N_EDGES // NW
CHUNK = 128
NCH = 80
E_PAD_W = NCH * CHUNK
ROWS_PER_TILE = N_PAD // NS
DR = N_PAD // 16

_MESH = dict(core_axis_name="c", subcore_axis_name="s")


def _make_sc_aggregate():
    mesh = plsc.VectorSubcoreMesh(**_MESH)

    @functools.partial(
        pl.kernel,
        out_type=jax.ShapeDtypeStruct((NC * N_PAD, D), jnp.float32),
        mesh=mesh,
        scratch_types=(
            pltpu.VMEM((CHUNK,), jnp.int32),
            pltpu.VMEM((CHUNK,), jnp.int32),
            pltpu.VMEM((CHUNK,), jnp.int32),
            pltpu.VMEM((CHUNK,), jnp.int32),
            pltpu.VMEM((CHUNK,), jnp.int32),
            pltpu.VMEM((CHUNK,), jnp.int32),
            pltpu.VMEM((CHUNK, D), jnp.float32),
            pltpu.VMEM((CHUNK, D), jnp.float32),
            pltpu.VMEM((CHUNK, D), jnp.float32),
            pltpu.VMEM_SHARED((N_PAD, D), jnp.float32),
            pltpu.SemaphoreType.DMA,
            pltpu.SemaphoreType.DMA,
            pltpu.SemaphoreType.DMA,
            pltpu.SemaphoreType.DMA,
        ),
    )
    def sc_agg(h_hbm, src_hbm, dst_hbm, z_hbm, out_hbm,
               src_v0, src_v1, src_v2, dst_v0, dst_v1, dst_v2,
               rows0, rows1, rows2, acc, sem_g0, sem_g1, sem_g2, sem_i):
        cid = lax.axis_index("c")
        sid = lax.axis_index("s")
        wid = cid * NS + sid
        ebase = wid * E_PAD_W

        r0 = sid * ROWS_PER_TILE
        pltpu.sync_copy(z_hbm.at[pl.ds(r0, ROWS_PER_TILE)],
                        acc.at[pl.ds(r0, ROWS_PER_TILE)])
        plsc.subcore_barrier()

        srcs = (src_v0, src_v1, src_v2)
        dsts = (dst_v0, dst_v1, dst_v2)
        rows = (rows0, rows1, rows2)
        sem_g = (sem_g0, sem_g1, sem_g2)
        NB = 3
        gd = [None] * NB
        pend = [None] * NB

        def load_idx(c):
            b = c % NB
            off = ebase + c * CHUNK
            i0 = pltpu.async_copy(src_hbm.at[pl.ds(off, CHUNK)],
                                  srcs[b], sem_i)
            i1 = pltpu.async_copy(dst_hbm.at[pl.ds(off, CHUNK)],
                                  dsts[b], sem_i)
            pend[b] = (i0, i1)

        def start_gather(c):
            b = c % NB
            pend[b][0].wait()
            pend[b][1].wait()
            gd[b] = pltpu.async_copy(h_hbm.at[srcs[b]], rows[b], sem_g[b])

        for c in range(NB):
            load_idx(c)
        start_gather(0)
        start_gather(1)
        for c in range(NCH):
            b = c % NB
            if c + 2 < NCH:
                start_gather(c + 2)
            gd[b].wait()
            pltpu.sync_copy(rows[b], acc.at[dsts[b]], add=True)
            if c + NB < NCH:
                load_idx(c + NB)

        plsc.subcore_barrier()

        obase = cid * N_PAD + r0
        pltpu.sync_copy(acc.at[pl.ds(r0, ROWS_PER_TILE)],
                        out_hbm.at[pl.ds(obase, ROWS_PER_TILE)])

    return sc_agg


def _make_sc_deg():
    mesh = plsc.VectorSubcoreMesh(**_MESH)

    @functools.partial(
        pl.kernel,
        out_type=jax.ShapeDtypeStruct((NW * DR, 16), jnp.float32),
        mesh=mesh,
        compiler_params=pltpu.CompilerParams(needs_layout_passes=False),
        scratch_types=(
            pltpu.VMEM((E_PER_W,), jnp.int32),
            pltpu.VMEM((DR, 16), jnp.float32),
        ),
    )
    def sc_deg(dst_hbm, z_hbm, out_hbm, dst_v, deg_v):
        cid = lax.axis_index("c")
        sid = lax.axis_index("s")
        wid = cid * NS + sid
        pltpu.sync_copy(dst_hbm.at[pl.ds(wid * E_PER_W, E_PER_W)], dst_v)
        pltpu.sync_copy(z_hbm, deg_v)
        ones16 = jnp.ones((16,), jnp.float32)

        def body(j, carry):
            d = dst_v[pl.ds(j * 16, 16)]
            plsc.addupdate_scatter(deg_v, [d >> 4, d & 15], ones16)
            return carry

        lax.fori_loop(0, E_PER_W // 16, body, 0)
        pltpu.sync_copy(deg_v, out_hbm.at[pl.ds(wid * DR, DR)])

    return sc_deg


_sc_agg = _make_sc_aggregate()
_sc_deg = _make_sc_deg()

def _linear1_body(h_ref, p0_ref, p1_ref, dp_ref, wt_ref, wb_ref,
                  b_ref, o_ref, di_ref):
    deg = jnp.sum(dp_ref[...], axis=0)
    dinv = (1.0 / jnp.maximum(deg, 1.0))[:, None]
    di_ref[...] = dinv
    agg = (p0_ref[...] + p1_ref[...]) * dinv
    acc = jnp.dot(h_ref[...], wt_ref[...], preferred_element_type=jnp.float32)
    acc = acc + jnp.dot(agg, wb_ref[...], preferred_element_type=jnp.float32)
    o_ref[...] = jnp.maximum(acc + b_ref[...], 0.0)


def _tc_linear1(h, p0, p1, degp, wt, wb, b):
    return pl.pallas_call(
        _linear1_body,
        out_shape=(jax.ShapeDtypeStruct((N_PAD, D), jnp.float32),
                   jax.ShapeDtypeStruct((N_PAD, 1), jnp.float32)),
    )(h, p0, p1, degp, wt, wb, b)


def _linear_body(relu, h_ref, p0_ref, p1_ref, di_ref, wt_ref, wb_ref,
                 b_ref, o_ref):
    agg = (p0_ref[...] + p1_ref[...]) * di_ref[...]
    acc = jnp.dot(h_ref[...], wt_ref[...], preferred_element_type=jnp.float32)
    acc = acc + jnp.dot(agg, wb_ref[...], preferred_element_type=jnp.float32)
    acc = acc + b_ref[...]
    if relu:
        acc = jnp.maximum(acc, 0.0)
    o_ref[...] = acc


def _tc_linear(h, p0, p1, dinv, wt, wb, b, relu):
    return pl.pallas_call(
        functools.partial(_linear_body, relu),
        out_shape=jax.ShapeDtypeStruct((N_PAD, D), jnp.float32),
    )(h, p0, p1, dinv, wt, wb, b)


def kernel(x, edge_index, W1, b1, W2, b2, W3, b3):
    e = edge_index.astype(jnp.int32)
    src, dst = e[0], e[1]
    h = jnp.pad(x, ((0, N_PAD - N_NODES), (0, 0)))
    zeros = jnp.zeros((N_PAD, D), jnp.float32)
    zerosd = jnp.zeros((DR, 16), jnp.float32)

    pad_tgt = N_NODES + (jnp.arange(E_PAD_W - E_PER_W, dtype=jnp.int32)
                         % (N_PAD - N_NODES))
    pad_blk = jnp.broadcast_to(pad_tgt, (NW, E_PAD_W - E_PER_W))

    def pad_chunks(v):
        v2 = jnp.concatenate([v.reshape(NW, E_PER_W), pad_blk], axis=1)
        return v2.reshape(NW * E_PAD_W)

    srcp, dstp = pad_chunks(src), pad_chunks(dst)

    degp = _sc_deg(dst, zerosd)

    def agg_layer(hh):
        pt = _sc_agg(hh, srcp, dstp, zeros)
        return pt[:N_PAD], pt[N_PAD:]

    p0, p1 = agg_layer(h)
    h1, dinv = _tc_linear1(h, p0, p1, degp.reshape(NW, N_PAD),
                           W1[:D], W1[D:], b1[None, :])
    a0, a1 = agg_layer(h1)
    h2 = _tc_linear(h1, a0, a1, dinv, W2[:D], W2[D:], b2[None, :], True)
    a0, a1 = agg_layer(h2)
    out = _tc_linear(h2, a0, a1, dinv, W3[:D], W3[D:], b3[None, :], False)
    return out[:N_NODES]


# speedup vs baseline: 1.0017x; 1.0017x over previous
import functools

import jax
import jax.numpy as jnp
from jax import lax
from jax.experimental import pallas as pl
from jax.experimental.pallas import tpu as pltpu
from jax.experimental.pallas import tpu_sc as plsc

N_NODES = 10000
N_PAD = 10112
N_EDGES = 320000
D = 128
NC = 2
NS = 16
NW = NC * NS
E_PER_W = N_EDGES // NW
CHUNK = 128
NCH = 80
E_PAD_W = NCH * CHUNK
ROWS_PER_TILE = N_PAD // NS
DR = N_PAD // 16

_MESH = dict(core_axis_name="c", subcore_axis_name="s")


def _make_sc_aggregate():
    mesh = plsc.VectorSubcoreMesh(**_MESH)

    @functools.partial(
        pl.kernel,
        out_type=jax.ShapeDtypeStruct((NC * N_PAD, D), jnp.float32),
        mesh=mesh,
        scratch_types=(
            pltpu.VMEM((CHUNK,), jnp.int32),
            pltpu.VMEM((CHUNK,), jnp.int32),
            pltpu.VMEM((CHUNK,), jnp.int32),
            pltpu.VMEM((CHUNK,), jnp.int32),
            pltpu.VMEM((CHUNK,), jnp.int32),
            pltpu.VMEM((CHUNK,), jnp.int32),
            pltpu.VMEM((CHUNK, D), jnp.float32),
            pltpu.VMEM((CHUNK, D), jnp.float32),
            pltpu.VMEM((CHUNK, D), jnp.float32),
            pltpu.VMEM_SHARED((N_PAD, D), jnp.float32),
            pltpu.SemaphoreType.DMA,
            pltpu.SemaphoreType.DMA,
            pltpu.SemaphoreType.DMA,
            pltpu.SemaphoreType.DMA,
        ),
    )
    def sc_agg(h_hbm, src_hbm, dst_hbm, z_hbm, out_hbm,
               src_v0, src_v1, src_v2, dst_v0, dst_v1, dst_v2,
               rows0, rows1, rows2, acc, sem_g0, sem_g1, sem_g2, sem_i):
        cid = lax.axis_index("c")
        sid = lax.axis_index("s")
        wid = cid * NS + sid
        ebase = wid * E_PAD_W

        r0 = sid * ROWS_PER_TILE
        pltpu.sync_copy(z_hbm.at[pl.ds(r0, ROWS_PER_TILE)],
                        acc.at[pl.ds(r0, ROWS_PER_TILE)])
        plsc.subcore_barrier()

        srcs = (src_v0, src_v1, src_v2)
        dsts = (dst_v0, dst_v1, dst_v2)
        rows = (rows0, rows1, rows2)
        sem_g = (sem_g0, sem_g1, sem_g2)
        NB = 3
        gd = [None] * NB
        pend = [None] * NB

        def load_idx(c):
            b = c % NB
            off = ebase + c * CHUNK
            i0 = pltpu.async_copy(src_hbm.at[pl.ds(off, CHUNK)],
                                  srcs[b], sem_i)
            i1 = pltpu.async_copy(dst_hbm.at[pl.ds(off, CHUNK)],
                                  dsts[b], sem_i)
            pend[b] = (i0, i1)

        def start_gather(c):
            b = c % NB
            pend[b][0].wait()
            pend[b][1].wait()
            gd[b] = pltpu.async_copy(h_hbm.at[srcs[b]], rows[b], sem_g[b])

        for c in range(NB):
            load_idx(c)
        start_gather(0)
        start_gather(1)
        for c in range(NCH):
            b = c % NB
            if c + 2 < NCH:
                start_gather(c + 2)
            gd[b].wait()
            pltpu.sync_copy(rows[b], acc.at[dsts[b]], add=True)
            if c + NB < NCH:
                load_idx(c + NB)

        plsc.subcore_barrier()

        obase = cid * N_PAD + r0
        pltpu.sync_copy(acc.at[pl.ds(r0, ROWS_PER_TILE)],
                        out_hbm.at[pl.ds(obase, ROWS_PER_TILE)])

    return sc_agg


def _make_sc_deg():
    mesh = plsc.VectorSubcoreMesh(**_MESH)

    @functools.partial(
        pl.kernel,
        out_type=jax.ShapeDtypeStruct((NW * DR, 16), jnp.float32),
        mesh=mesh,
        compiler_params=pltpu.CompilerParams(needs_layout_passes=False),
        scratch_types=(
            pltpu.VMEM((E_PER_W,), jnp.int32),
            pltpu.VMEM((DR, 16), jnp.float32),
        ),
    )
    def sc_deg(dst_hbm, z_hbm, out_hbm, dst_v, deg_v):
        cid = lax.axis_index("c")
        sid = lax.axis_index("s")
        wid = cid * NS + sid
        pltpu.sync_copy(dst_hbm.at[pl.ds(wid * E_PER_W, E_PER_W)], dst_v)
        pltpu.sync_copy(z_hbm, deg_v)
        ones16 = jnp.ones((16,), jnp.float32)

        def body(j, carry):
            d = dst_v[pl.ds(j * 16, 16)]
            plsc.addupdate_scatter(deg_v, [d >> 4, d & 15], ones16)
            return carry

        lax.fori_loop(0, E_PER_W // 16, body, 0)
        pltpu.sync_copy(deg_v, out_hbm.at[pl.ds(wid * DR, DR)])

    return sc_deg


_sc_agg = _make_sc_aggregate()
_sc_deg = _make_sc_deg()

ROW_BLK = 2528
_blk = lambda r, c: pl.BlockSpec((r, c), lambda i: (i, 0))
_full = lambda r, c: pl.BlockSpec((r, c), lambda i: (0, 0))


def _linear1_body(h_ref, p0_ref, p1_ref, dp_ref, wt_ref, wb_ref,
                  b_ref, o_ref, di_ref):
    deg = jnp.sum(dp_ref[...], axis=1)
    dinv = (1.0 / jnp.maximum(deg, 1.0))[:, None]
    di_ref[...] = dinv
    agg = (p0_ref[...] + p1_ref[...]) * dinv
    acc = jnp.dot(h_ref[...], wt_ref[...], preferred_element_type=jnp.float32)
    acc = acc + jnp.dot(agg, wb_ref[...], preferred_element_type=jnp.float32)
    o_ref[...] = jnp.maximum(acc + b_ref[...], 0.0)


def _tc_linear1(h, p0, p1, degpt, wt, wb, b):
    return pl.pallas_call(
        _linear1_body,
        grid=(N_PAD // ROW_BLK,),
        in_specs=[_blk(ROW_BLK, D), _blk(ROW_BLK, D), _blk(ROW_BLK, D),
                  _blk(ROW_BLK, NW),
                  _full(D, D), _full(D, D), _full(1, D)],
        out_specs=(_blk(ROW_BLK, D), _blk(ROW_BLK, 1)),
        out_shape=(jax.ShapeDtypeStruct((N_PAD, D), jnp.float32),
                   jax.ShapeDtypeStruct((N_PAD, 1), jnp.float32)),
    )(h, p0, p1, degpt, wt, wb, b)


def _linear_body(relu, h_ref, p0_ref, p1_ref, di_ref, wt_ref, wb_ref,
                 b_ref, o_ref):
    agg = (p0_ref[...] + p1_ref[...]) * di_ref[...]
    acc = jnp.dot(h_ref[...], wt_ref[...], preferred_element_type=jnp.float32)
    acc = acc + jnp.dot(agg, wb_ref[...], preferred_element_type=jnp.float32)
    acc = acc + b_ref[...]
    if relu:
        acc = jnp.maximum(acc, 0.0)
    o_ref[...] = acc


def _tc_linear(h, p0, p1, dinv, wt, wb, b, relu):
    return pl.pallas_call(
        functools.partial(_linear_body, relu),
        grid=(N_PAD // ROW_BLK,),
        in_specs=[_blk(ROW_BLK, D), _blk(ROW_BLK, D), _blk(ROW_BLK, D),
                  _blk(ROW_BLK, 1),
                  _full(D, D), _full(D, D), _full(1, D)],
        out_specs=_blk(ROW_BLK, D),
        out_shape=jax.ShapeDtypeStruct((N_PAD, D), jnp.float32),
    )(h, p0, p1, dinv, wt, wb, b)


def kernel(x, edge_index, W1, b1, W2, b2, W3, b3):
    e = edge_index.astype(jnp.int32)
    src, dst = e[0], e[1]
    h = jnp.pad(x, ((0, N_PAD - N_NODES), (0, 0)))
    zeros = jnp.zeros((N_PAD, D), jnp.float32)
    zerosd = jnp.zeros((DR, 16), jnp.float32)

    pad_tgt = N_NODES + (jnp.arange(E_PAD_W - E_PER_W, dtype=jnp.int32)
                         % (N_PAD - N_NODES))
    pad_blk = jnp.broadcast_to(pad_tgt, (NW, E_PAD_W - E_PER_W))

    def pad_chunks(v):
        v2 = jnp.concatenate([v.reshape(NW, E_PER_W), pad_blk], axis=1)
        return v2.reshape(NW * E_PAD_W)

    srcp, dstp = pad_chunks(src), pad_chunks(dst)

    degp = _sc_deg(dst, zerosd)

    def agg_layer(hh):
        pt = _sc_agg(hh, srcp, dstp, zeros)
        return pt[:N_PAD], pt[N_PAD:]

    p0, p1 = agg_layer(h)
    h1, dinv = _tc_linear1(h, p0, p1, degp.reshape(NW, N_PAD).T,
                           W1[:D], W1[D:], b1[None, :])
    a0, a1 = agg_layer(h1)
    h2 = _tc_linear(h1, a0, a1, dinv, W2[:D], W2[D:], b2[None, :], True)
    a0, a1 = agg_layer(h2)
    out = _tc_linear(h2, a0, a1, dinv, W3[:D], W3[D:], b3[None, :], False)
    return out[:N_NODES]


# speedup vs baseline: 1.1606x; 1.1587x over previous
import functools

import jax
import jax.numpy as jnp
from jax import lax
from jax.experimental import pallas as pl
from jax.experimental.pallas import tpu as pltpu
from jax.experimental.pallas import tpu_sc as plsc

N_NODES = 10000
N_PAD = 10112
N_EDGES = 320000
D = 128
NC = 2
NS = 16
NW = NC * NS
E_PER_W = N_EDGES // NW
CHUNK = 128
NCH = 80
E_PAD_W = NCH * CHUNK
ROWS_PER_TILE = N_PAD // NS
DR = N_PAD // 16

_MESH = dict(core_axis_name="c", subcore_axis_name="s")


def _make_sc_aggregate():
    mesh = plsc.VectorSubcoreMesh(**_MESH)

    @functools.partial(
        pl.kernel,
        out_type=jax.ShapeDtypeStruct((NC * N_PAD, D), jnp.float32),
        mesh=mesh,
        scratch_types=(
            pltpu.VMEM((CHUNK,), jnp.int32),
            pltpu.VMEM((CHUNK,), jnp.int32),
            pltpu.VMEM((CHUNK,), jnp.int32),
            pltpu.VMEM((CHUNK,), jnp.int32),
            pltpu.VMEM((CHUNK,), jnp.int32),
            pltpu.VMEM((CHUNK,), jnp.int32),
            pltpu.VMEM((CHUNK, D), jnp.float32),
            pltpu.VMEM((CHUNK, D), jnp.float32),
            pltpu.VMEM((CHUNK, D), jnp.float32),
            pltpu.VMEM_SHARED((N_PAD, D), jnp.float32),
            pltpu.SemaphoreType.DMA,
            pltpu.SemaphoreType.DMA,
            pltpu.SemaphoreType.DMA,
            pltpu.SemaphoreType.DMA,
            pltpu.SemaphoreType.DMA,
            pltpu.SemaphoreType.DMA,
            pltpu.SemaphoreType.DMA,
        ),
    )
    def sc_agg(h_hbm, src_hbm, dst_hbm, z_hbm, out_hbm,
               src_v0, src_v1, src_v2, dst_v0, dst_v1, dst_v2,
               rows0, rows1, rows2, acc,
               sem_g0, sem_g1, sem_g2, sem_is, sem_id0, sem_id1, sem_s):
        cid = lax.axis_index("c")
        sid = lax.axis_index("s")
        wid = cid * NS + sid
        ebase = wid * E_PAD_W

        r0 = sid * ROWS_PER_TILE
        pltpu.sync_copy(z_hbm.at[pl.ds(r0, ROWS_PER_TILE)],
                        acc.at[pl.ds(r0, ROWS_PER_TILE)])
        plsc.subcore_barrier()

        srcs = (src_v0, src_v1, src_v2)
        dsts = (dst_v0, dst_v1, dst_v2)
        rows = (rows0, rows1, rows2)
        sem_g = (sem_g0, sem_g1, sem_g2)
        sem_id = (sem_id0, sem_id1)
        NB = 3
        gd = [None] * NB
        pend_src = [None] * NB
        pend_dst = [None] * NB

        def load_src(c):
            off = ebase + c * CHUNK
            pend_src[c % NB] = pltpu.async_copy(
                src_hbm.at[pl.ds(off, CHUNK)], srcs[c % NB], sem_is)

        def load_dst(c):
            off = ebase + c * CHUNK
            pend_dst[c % NB] = pltpu.async_copy(
                dst_hbm.at[pl.ds(off, CHUNK)], dsts[c % NB], sem_id[c % 2])

        def start_gather(c):
            b = c % NB
            pend_src[b].wait()
            gd[b] = pltpu.async_copy(h_hbm.at[srcs[b]], rows[b], sem_g[b])

        for c in range(NB):
            load_src(c)
        load_dst(0)
        load_dst(1)
        start_gather(0)
        start_gather(1)
        sd = None
        for c in range(NCH):
            b = c % NB
            if sd is not None:
                sd.wait()
            if c + 2 < NCH:
                start_gather(c + 2)
            gd[b].wait()
            if c + 3 < NCH:
                load_src(c + 3)
            pend_dst[b].wait()
            sd = pltpu.async_copy(rows[b], acc.at[dsts[b]], sem_s,
                                  add=True)
            if c + 2 < NCH:
                load_dst(c + 2)
        sd.wait()

        plsc.subcore_barrier()

        obase = cid * N_PAD + r0
        pltpu.sync_copy(acc.at[pl.ds(r0, ROWS_PER_TILE)],
                        out_hbm.at[pl.ds(obase, ROWS_PER_TILE)])

    return sc_agg


def _make_sc_deg():
    mesh = plsc.VectorSubcoreMesh(**_MESH)

    @functools.partial(
        pl.kernel,
        out_type=jax.ShapeDtypeStruct((NW * DR, 16), jnp.float32),
        mesh=mesh,
        compiler_params=pltpu.CompilerParams(needs_layout_passes=False),
        scratch_types=(
            pltpu.VMEM((E_PER_W,), jnp.int32),
            pltpu.VMEM((DR, 16), jnp.float32),
        ),
    )
    def sc_deg(dst_hbm, z_hbm, out_hbm, dst_v, deg_v):
        cid = lax.axis_index("c")
        sid = lax.axis_index("s")
        wid = cid * NS + sid
        pltpu.sync_copy(dst_hbm.at[pl.ds(wid * E_PER_W, E_PER_W)], dst_v)
        pltpu.sync_copy(z_hbm, deg_v)
        ones16 = jnp.ones((16,), jnp.float32)

        def body(j, carry):
            d = dst_v[pl.ds(j * 16, 16)]
            plsc.addupdate_scatter(deg_v, [d >> 4, d & 15], ones16)
            return carry

        lax.fori_loop(0, E_PER_W // 16, body, 0)
        pltpu.sync_copy(deg_v, out_hbm.at[pl.ds(wid * DR, DR)])

    return sc_deg


_sc_agg = _make_sc_aggregate()
_sc_deg = _make_sc_deg()

ROW_BLK = 2528
_blk = lambda r, c: pl.BlockSpec((r, c), lambda i: (i, 0))
_full = lambda r, c: pl.BlockSpec((r, c), lambda i: (0, 0))


def _linear1_body(h_ref, p0_ref, p1_ref, dp_ref, wt_ref, wb_ref,
                  b_ref, o_ref, di_ref):
    deg = jnp.sum(dp_ref[...], axis=1)
    dinv = (1.0 / jnp.maximum(deg, 1.0))[:, None]
    di_ref[...] = dinv
    agg = (p0_ref[...] + p1_ref[...]) * dinv
    acc = jnp.dot(h_ref[...], wt_ref[...], preferred_element_type=jnp.float32)
    acc = acc + jnp.dot(agg, wb_ref[...], preferred_element_type=jnp.float32)
    o_ref[...] = jnp.maximum(acc + b_ref[...], 0.0)


def _tc_linear1(h, p0, p1, degpt, wt, wb, b):
    return pl.pallas_call(
        _linear1_body,
        grid=(N_PAD // ROW_BLK,),
        in_specs=[_blk(ROW_BLK, D), _blk(ROW_BLK, D), _blk(ROW_BLK, D),
                  _blk(ROW_BLK, NW),
                  _full(D, D), _full(D, D), _full(1, D)],
        out_specs=(_blk(ROW_BLK, D), _blk(ROW_BLK, 1)),
        out_shape=(jax.ShapeDtypeStruct((N_PAD, D), jnp.float32),
                   jax.ShapeDtypeStruct((N_PAD, 1), jnp.float32)),
    )(h, p0, p1, degpt, wt, wb, b)


def _linear_body(relu, h_ref, p0_ref, p1_ref, di_ref, wt_ref, wb_ref,
                 b_ref, o_ref):
    agg = (p0_ref[...] + p1_ref[...]) * di_ref[...]
    acc = jnp.dot(h_ref[...], wt_ref[...], preferred_element_type=jnp.float32)
    acc = acc + jnp.dot(agg, wb_ref[...], preferred_element_type=jnp.float32)
    acc = acc + b_ref[...]
    if relu:
        acc = jnp.maximum(acc, 0.0)
    o_ref[...] = acc


def _tc_linear(h, p0, p1, dinv, wt, wb, b, relu):
    return pl.pallas_call(
        functools.partial(_linear_body, relu),
        grid=(N_PAD // ROW_BLK,),
        in_specs=[_blk(ROW_BLK, D), _blk(ROW_BLK, D), _blk(ROW_BLK, D),
                  _blk(ROW_BLK, 1),
                  _full(D, D), _full(D, D), _full(1, D)],
        out_specs=_blk(ROW_BLK, D),
        out_shape=jax.ShapeDtypeStruct((N_PAD, D), jnp.float32),
    )(h, p0, p1, dinv, wt, wb, b)


def kernel(x, edge_index, W1, b1, W2, b2, W3, b3):
    e = edge_index.astype(jnp.int32)
    src, dst = e[0], e[1]
    h = jnp.pad(x, ((0, N_PAD - N_NODES), (0, 0)))
    zeros = jnp.zeros((N_PAD, D), jnp.float32)
    zerosd = jnp.zeros((DR, 16), jnp.float32)

    pad_tgt = N_NODES + (jnp.arange(E_PAD_W - E_PER_W, dtype=jnp.int32)
                         % (N_PAD - N_NODES))
    pad_blk = jnp.broadcast_to(pad_tgt, (NW, E_PAD_W - E_PER_W))

    def pad_chunks(v):
        v2 = jnp.concatenate([v.reshape(NW, E_PER_W), pad_blk], axis=1)
        return v2.reshape(NW * E_PAD_W)

    srcp, dstp = pad_chunks(src), pad_chunks(dst)

    degp = _sc_deg(dst, zerosd)

    def agg_layer(hh):
        pt = _sc_agg(hh, srcp, dstp, zeros)
        return pt[:N_PAD], pt[N_PAD:]

    p0, p1 = agg_layer(h)
    h1, dinv = _tc_linear1(h, p0, p1, degp.reshape(NW, N_PAD).T,
                           W1[:D], W1[D:], b1[None, :])
    a0, a1 = agg_layer(h1)
    h2 = _tc_linear(h1, a0, a1, dinv, W2[:D], W2[D:], b2[None, :], True)
    a0, a1 = agg_layer(h2)
    out = _tc_linear(h2, a0, a1, dinv, W3[:D], W3[D:], b3[None, :], False)
    return out[:N_NODES]
